# Initial kernel scaffold; baseline (speedup 1.0000x reference)
import jax
import jax.numpy as jnp
from jax.experimental import pallas as pl


def kernel(positions, velocities, features, indices, new_positions, new_velocities, new_features):
    raise NotImplementedError("write your pallas kernel here")


# speedup vs baseline: 52.8263x; 52.8263x over previous
import functools

import jax
import jax.numpy as jnp
from jax import lax
from jax.experimental import pallas as pl
from jax.experimental.pallas import tpu as pltpu
from jax.experimental.pallas import tpu_sc as plsc

_K = 16384
_M = 1000000
_D = 70
_DP = 80
_NC = 2
_NS = 16
_NW = _NC * _NS
_TN = _M // _NW
_L = 16
_C = 128
_NEG = -(2**31)


@jax.jit
def _sc_update_gather(indices, newcat):
    mesh = plsc.VectorSubcoreMesh(
        core_axis_name="c", subcore_axis_name="s",
        num_cores=_NC, num_subcores=_NS)

    @functools.partial(
        pl.kernel,
        out_type=jax.ShapeDtypeStruct((_K, _DP), jnp.float32),
        mesh=mesh,
        compiler_params=pltpu.CompilerParams(needs_layout_passes=False,
                                             use_tc_tiling_on_sc=False),
        scratch_types=[
            pltpu.VMEM((_K,), jnp.int32),
            pltpu.VMEM((_TN,), jnp.int32),
            pltpu.VMEM((_K,), jnp.int32),
            pltpu.VMEM((_K,), jnp.int32),
            pltpu.VMEM((_K,), jnp.int32),
            pltpu.VMEM((_C, _DP), jnp.float32),
            pltpu.VMEM((_C,), jnp.int32),
            pltpu.SemaphoreType.DMA,
        ],
    )
    def body(idx_hbm, newcat_hbm, out_hbm,
             idx_v, table_v, comp_l, comp_kk, w_v, rows_v, kch_v, sem):
        wid = lax.axis_index("s") * _NC + lax.axis_index("c")
        base = (wid * _TN).astype(jnp.int32)
        lanes = lax.iota(jnp.int32, _L)

        pltpu.sync_copy(idx_hbm, idx_v)

        def scan_body(i, off):
            idxv = idx_v[pl.ds(i * _L, _L)]
            loc = idxv - base
            owned = (loc >= 0) & (loc < _TN)
            loc_c = jnp.clip(loc, 0, _TN - 1)
            kv = lanes + i * _L
            plsc.store_scatter(table_v, [loc_c], kv, mask=owned)
            prefix = plsc.cumsum(owned.astype(jnp.int32))
            addr = off + prefix - 1
            addr_c = jnp.clip(addr, 0, _K - 1)
            plsc.store_scatter(comp_l, [addr_c], loc_c, mask=owned)
            plsc.store_scatter(comp_kk, [addr_c], kv, mask=owned)
            return off + jnp.max(prefix)

        cnt = lax.fori_loop(0, _K // _L, scan_body, jnp.int32(0))

        nch = (cnt + _C - 1) // _C
        l0 = comp_l[pl.ds(0, _L)]
        k0 = comp_kk[pl.ds(0, _L)]
        l0b = jnp.full((_L,), jnp.max(jnp.where(lanes == 0, l0, _NEG)),
                       jnp.int32)
        k0b = jnp.full((_L,), jnp.max(jnp.where(lanes == 0, k0, _NEG)),
                       jnp.int32)

        def pad_body(j, _):
            pos = j * _L + lanes
            m = pos >= cnt
            cur_l = comp_l[pl.ds(j * _L, _L)]
            cur_k = comp_kk[pl.ds(j * _L, _L)]
            comp_l[pl.ds(j * _L, _L)] = jnp.where(m, l0b, cur_l)
            comp_kk[pl.ds(j * _L, _L)] = jnp.where(m, k0b, cur_k)
            return 0

        nv = nch * (_C // _L)
        lax.fori_loop(cnt // _L, nv, pad_body, 0)

        def corr_body(i, _):
            lv = comp_l[pl.ds(i * _L, _L)]
            kvv = comp_kk[pl.ds(i * _L, _L)]
            wv = plsc.load_gather(table_v, [lv])
            plsc.store_scatter(table_v, [lv], kvv, mask=kvv > wv)
            return 0

        def round_body(r, _):
            lax.fori_loop(0, nv, corr_body, 0)
            return 0

        lax.fori_loop(0, 15, round_body, 0)

        def wg_body(i, _):
            lv = comp_l[pl.ds(i * _L, _L)]
            w_v[pl.ds(i * _L, _L)] = plsc.load_gather(table_v, [lv])
            return 0

        lax.fori_loop(0, nv, wg_body, 0)

        def row_body(ch, _):
            pltpu.sync_copy(newcat_hbm.at[w_v.at[pl.ds(ch * _C, _C)]],
                            rows_v)

            def cp_body(j, _):
                kch_v[pl.ds(j * _L, _L)] = comp_kk[pl.ds(ch * _C + j * _L, _L)]
                return 0

            lax.fori_loop(0, _C // _L, cp_body, 0)
            pltpu.sync_copy(rows_v, out_hbm.at[kch_v])
            return 0

        lax.fori_loop(0, nch, row_body, 0)

    return body(indices, newcat)


def kernel(positions, velocities, features, indices,
           new_positions, new_velocities, new_features):
    del positions, velocities, features
    newcat = jnp.concatenate(
        [new_positions, new_velocities, new_features,
         jnp.zeros((_K, _DP - _D), jnp.float32)], axis=1)
    out = _sc_update_gather(indices.astype(jnp.int32), newcat)
    return out[:, :_D]


# speedup vs baseline: 56.4822x; 1.0692x over previous
import functools

import jax
import jax.numpy as jnp
from jax import lax
from jax.experimental import pallas as pl
from jax.experimental.pallas import tpu as pltpu
from jax.experimental.pallas import tpu_sc as plsc

_K = 16384
_M = 1000000
_D = 70
_DP = 80
_NC = 2
_NS = 16
_NW = _NC * _NS
_TN = _M // _NW
_L = 16
_C = 128
_KB = 14
_NEG = -(2**31)


@jax.jit
def _sc_update_gather(indices, newcat):
    mesh = plsc.VectorSubcoreMesh(
        core_axis_name="c", subcore_axis_name="s",
        num_cores=_NC, num_subcores=_NS)

    @functools.partial(
        pl.kernel,
        out_type=jax.ShapeDtypeStruct((_K, _DP), jnp.float32),
        mesh=mesh,
        compiler_params=pltpu.CompilerParams(needs_layout_passes=False,
                                             use_tc_tiling_on_sc=False),
        scratch_types=[
            pltpu.VMEM((_K,), jnp.int32),
            pltpu.VMEM((_TN + _L,), jnp.int32),
            pltpu.VMEM((_K,), jnp.int32),
            pltpu.VMEM((_K,), jnp.int32),
            pltpu.VMEM((_C, _DP), jnp.float32),
            pltpu.VMEM((_C,), jnp.int32),
            pltpu.SemaphoreType.DMA,
        ],
    )
    def body(idx_hbm, newcat_hbm, out_hbm,
             idx_v, table_v, comp_v, w_v, rows_v, kch_v, sem):
        wid = lax.axis_index("s") * _NC + lax.axis_index("c")
        base = (wid * _TN).astype(jnp.int32)
        lanes = lax.iota(jnp.int32, _L)

        pltpu.sync_copy(idx_hbm, idx_v)

        def scan_body(i, off):
            idxv = idx_v[pl.ds(i * _L, _L)]
            loc = idxv - base
            owned = (loc >= 0) & (loc < _TN)
            keyv = jnp.clip(loc, 0, _TN - 1) * (2**_KB) + lanes + i * _L
            prefix = plsc.cumsum(owned.astype(jnp.int32))
            addr_c = jnp.clip(off + prefix - 1, 0, _K - 1)
            plsc.store_scatter(comp_v, [addr_c], keyv, mask=owned)
            return off + jnp.max(prefix)

        cnt = lax.fori_loop(0, _K // _L, scan_body, jnp.int32(0))

        nch = (cnt + _C - 1) // _C
        key0 = comp_v[pl.ds(0, _L)]
        key0b = jnp.full((_L,), jnp.max(jnp.where(lanes == 0, key0, _NEG)),
                         jnp.int32)

        def pad_body(j, _):
            pos = j * _L + lanes
            cur = comp_v[pl.ds(j * _L, _L)]
            comp_v[pl.ds(j * _L, _L)] = jnp.where(pos >= cnt, key0b, cur)
            return 0

        nv = nch * (_C // _L)
        lax.fori_loop(cnt // _L, nv, pad_body, 0)

        sent = jnp.int32(_TN * (2**_KB))
        shift1 = jnp.minimum(lanes + 1, _L - 1)

        def win_body(i, _):
            keyv = comp_v[pl.ds(i * _L, _L)]
            keyv = jnp.where(i * _L + lanes < cnt, keyv, sent)
            skey = plsc.sort_key_val(keyv, keyv)[0]
            sloc = skey // (2**_KB)
            snxt = sloc.at[shift1].get(mode="promise_in_bounds")
            last = (lanes == _L - 1) | (sloc != snxt)
            plsc.store_scatter(table_v, [sloc], skey & (2**_KB - 1),
                               mask=last)
            return 0

        lax.fori_loop(0, (cnt + _L - 1) // _L, win_body, 0)

        def wg_body(i, _):
            lv = comp_v[pl.ds(i * _L, _L)] // (2**_KB)
            w_v[pl.ds(i * _L, _L)] = plsc.load_gather(table_v, [lv])
            return 0

        lax.fori_loop(0, nv, wg_body, 0)

        def row_body(ch, _):
            pltpu.sync_copy(newcat_hbm.at[w_v.at[pl.ds(ch * _C, _C)]],
                            rows_v)

            def cp_body(j, _):
                kch_v[pl.ds(j * _L, _L)] = (
                    comp_v[pl.ds(ch * _C + j * _L, _L)] & (2**_KB - 1))
                return 0

            lax.fori_loop(0, _C // _L, cp_body, 0)
            pltpu.sync_copy(rows_v, out_hbm.at[kch_v])
            return 0

        lax.fori_loop(0, nch, row_body, 0)

    return body(indices, newcat)


def kernel(positions, velocities, features, indices,
           new_positions, new_velocities, new_features):
    del positions, velocities, features
    newcat = jnp.concatenate(
        [new_positions, new_velocities, new_features,
         jnp.zeros((_K, _DP - _D), jnp.float32)], axis=1)
    out = _sc_update_gather(indices.astype(jnp.int32), newcat)
    return out[:, :_D]


# speedup vs baseline: 67.4163x; 1.1936x over previous
import functools

import jax
import jax.numpy as jnp
from jax import lax
from jax.experimental import pallas as pl
from jax.experimental.pallas import tpu as pltpu
from jax.experimental.pallas import tpu_sc as plsc

_K = 16384
_M = 1000000
_D = 70
_DP = 128
_NC = 2
_NS = 16
_NW = _NC * _NS
_TN = _M // _NW
_L = 16
_C = 128
_KB = 14
_NEG = -(2**31)


@jax.jit
def _sc_update_gather(indices, newcat):
    mesh = plsc.VectorSubcoreMesh(
        core_axis_name="c", subcore_axis_name="s",
        num_cores=_NC, num_subcores=_NS)

    @functools.partial(
        pl.kernel,
        out_type=jax.ShapeDtypeStruct((_K, _DP), jnp.float32),
        mesh=mesh,
        compiler_params=pltpu.CompilerParams(needs_layout_passes=False,
                                             use_tc_tiling_on_sc=False),
        scratch_types=[
            pltpu.VMEM((_K,), jnp.int32),
            pltpu.VMEM((_TN + _L,), jnp.int32),
            pltpu.VMEM((_K,), jnp.int32),
            pltpu.VMEM((_K,), jnp.int32),
            pltpu.VMEM((_C, _DP), jnp.float32),
            pltpu.VMEM((_C,), jnp.int32),
            pltpu.SemaphoreType.DMA,
        ],
    )
    def body(idx_hbm, newcat_hbm, out_hbm,
             idx_v, table_v, comp_v, w_v, rows_v, kch_v, sem):
        wid = lax.axis_index("s") * _NC + lax.axis_index("c")
        base = (wid * _TN).astype(jnp.int32)
        lanes = lax.iota(jnp.int32, _L)

        pltpu.sync_copy(idx_hbm, idx_v)

        def scan_body(i, off):
            for u in range(4):
                j = i * 4 + u
                idxv = idx_v[pl.ds(j * _L, _L)]
                loc = idxv - base
                owned = (loc >= 0) & (loc < _TN)
                keyv = jnp.clip(loc, 0, _TN - 1) * (2**_KB) + lanes + j * _L
                prefix = plsc.cumsum(owned.astype(jnp.int32))
                addr_c = jnp.clip(off + prefix - 1, 0, _K - 1)
                plsc.store_scatter(comp_v, [addr_c], keyv, mask=owned)
                off = off + jnp.max(prefix)
            return off

        cnt = lax.fori_loop(0, _K // _L // 4, scan_body, jnp.int32(0))

        nch = (cnt + _C - 1) // _C
        key0 = comp_v[pl.ds(0, _L)]
        key0b = jnp.full((_L,), jnp.max(jnp.where(lanes == 0, key0, _NEG)),
                         jnp.int32)

        def pad_body(j, _):
            pos = j * _L + lanes
            cur = comp_v[pl.ds(j * _L, _L)]
            comp_v[pl.ds(j * _L, _L)] = jnp.where(pos >= cnt, key0b, cur)
            return 0

        nv = nch * (_C // _L)
        lax.fori_loop(cnt // _L, nv, pad_body, 0)

        sent = jnp.int32(_TN * (2**_KB))
        shift1 = jnp.minimum(lanes + 1, _L - 1)

        def win_body(i, _):
            keyv = comp_v[pl.ds(i * _L, _L)]
            keyv = jnp.where(i * _L + lanes < cnt, keyv, sent)
            skey = plsc.sort_key_val(keyv, keyv)[0]
            sloc = skey // (2**_KB)
            snxt = sloc.at[shift1].get(mode="promise_in_bounds")
            last = (lanes == _L - 1) | (sloc != snxt)
            plsc.store_scatter(table_v, [sloc], skey & (2**_KB - 1),
                               mask=last)
            return 0

        lax.fori_loop(0, (cnt + _L - 1) // _L, win_body, 0)

        def wg_body(i, _):
            lv = comp_v[pl.ds(i * _L, _L)] // (2**_KB)
            w_v[pl.ds(i * _L, _L)] = plsc.load_gather(table_v, [lv])
            return 0

        lax.fori_loop(0, nv, wg_body, 0)

        def row_body(ch, _):
            pltpu.sync_copy(newcat_hbm.at[w_v.at[pl.ds(ch * _C, _C)]],
                            rows_v)

            def cp_body(j, _):
                kch_v[pl.ds(j * _L, _L)] = (
                    comp_v[pl.ds(ch * _C + j * _L, _L)] & (2**_KB - 1))
                return 0

            lax.fori_loop(0, _C // _L, cp_body, 0)
            pltpu.sync_copy(rows_v, out_hbm.at[kch_v])
            return 0

        lax.fori_loop(0, nch, row_body, 0)

    return body(indices, newcat)


def kernel(positions, velocities, features, indices,
           new_positions, new_velocities, new_features):
    del positions, velocities, features
    newcat = jnp.concatenate(
        [new_positions, new_velocities, new_features,
         jnp.zeros((_K, _DP - _D), jnp.float32)], axis=1)
    out = _sc_update_gather(indices.astype(jnp.int32), newcat)
    return out[:, :_D]


# speedup vs baseline: 74.4247x; 1.1040x over previous
import functools

import jax
import jax.numpy as jnp
from jax import lax
from jax.experimental import pallas as pl
from jax.experimental.pallas import tpu as pltpu
from jax.experimental.pallas import tpu_sc as plsc

_K = 16384
_M = 1000000
_D = 70
_DP = 128
_NC = 2
_NS = 16
_NW = _NC * _NS
_TN = _M // _NW
_L = 16
_C = 128
_KB = 14
_NEG = -(2**31)


@jax.jit
def _sc_update_gather(indices, newcat):
    mesh = plsc.VectorSubcoreMesh(
        core_axis_name="c", subcore_axis_name="s",
        num_cores=_NC, num_subcores=_NS)

    @functools.partial(
        pl.kernel,
        out_type=jax.ShapeDtypeStruct((_K, _DP), jnp.float32),
        mesh=mesh,
        compiler_params=pltpu.CompilerParams(needs_layout_passes=False,
                                             use_tc_tiling_on_sc=False),
        scratch_types=[
            pltpu.VMEM((_K,), jnp.int32),
            pltpu.VMEM((_TN + _L,), jnp.int32),
            pltpu.VMEM((_K,), jnp.int32),
            pltpu.VMEM((_K,), jnp.int32),
            pltpu.VMEM((2, _C, _DP), jnp.float32),
            pltpu.VMEM((2, _C), jnp.int32),
            pltpu.SemaphoreType.DMA,
            pltpu.SemaphoreType.DMA,
        ],
    )
    def body(idx_hbm, newcat_hbm, out_hbm,
             idx_v, table_v, comp_v, w_v, rows_v, kch_v, gsem, ssem):
        wid = lax.axis_index("s") * _NC + lax.axis_index("c")
        base = (wid * _TN).astype(jnp.int32)
        lanes = lax.iota(jnp.int32, _L)

        pltpu.sync_copy(idx_hbm, idx_v)

        def scan_body(i, off):
            for u in range(4):
                j = i * 4 + u
                idxv = idx_v[pl.ds(j * _L, _L)]
                loc = idxv - base
                owned = (loc >= 0) & (loc < _TN)
                keyv = jnp.clip(loc, 0, _TN - 1) * (2**_KB) + lanes + j * _L
                plsc.store_compressed(comp_v.at[pl.ds(off, _L)], keyv,
                                      mask=owned)
                off = off + plsc.all_reduce_population_count(owned)[0]
            return off

        cnt = lax.fori_loop(0, _K // _L // 4, scan_body, jnp.int32(0))

        nch = (cnt + _C - 1) // _C
        key0 = comp_v[pl.ds(0, _L)]
        key0b = jnp.full((_L,), jnp.max(jnp.where(lanes == 0, key0, _NEG)),
                         jnp.int32)

        def pad_body(j, _):
            pos = j * _L + lanes
            cur = comp_v[pl.ds(j * _L, _L)]
            comp_v[pl.ds(j * _L, _L)] = jnp.where(pos >= cnt, key0b, cur)
            return 0

        nv = nch * (_C // _L)
        lax.fori_loop(cnt // _L, nv, pad_body, 0)

        sent = jnp.int32(_TN * (2**_KB))
        shift1 = jnp.minimum(lanes + 1, _L - 1)

        def win_body(i, _):
            keyv = comp_v[pl.ds(i * _L, _L)]
            keyv = jnp.where(i * _L + lanes < cnt, keyv, sent)
            skey = plsc.sort_key_val(keyv, keyv)[0]
            sloc = skey // (2**_KB)
            snxt = sloc.at[shift1].get(mode="promise_in_bounds")
            last = (lanes == _L - 1) | (sloc != snxt)
            plsc.store_scatter(table_v, [sloc], skey & (2**_KB - 1),
                               mask=last)
            return 0

        lax.fori_loop(0, (cnt + _L - 1) // _L, win_body, 0)

        def wg_body(i, _):
            lv = comp_v[pl.ds(i * _L, _L)] // (2**_KB)
            w_v[pl.ds(i * _L, _L)] = plsc.load_gather(table_v, [lv])
            return 0

        lax.fori_loop(0, nv, wg_body, 0)

        def fill_kch(ch, b):
            def cp_body(j, _):
                kch_v.at[b][pl.ds(j * _L, _L)] = (
                    comp_v[pl.ds(ch * _C + j * _L, _L)] & (2**_KB - 1))
                return 0

            lax.fori_loop(0, _C // _L, cp_body, 0)

        def gather_start(ch, b):
            return pltpu.async_copy(
                newcat_hbm.at[w_v.at[pl.ds(ch * _C, _C)]],
                rows_v.at[b], gsem)

        @pl.when(nch > 0)
        def _():
            gather_start(0, 0).wait()
            fill_kch(0, 0)

            def row_body(ch, _):
                b = ch % 2
                g = gather_start(ch + 1, 1 - b)
                pltpu.async_copy(rows_v.at[b], out_hbm.at[kch_v.at[b]],
                                 ssem).wait()
                g.wait()
                fill_kch(ch + 1, 1 - b)
                return 0

            lax.fori_loop(0, nch - 1, row_body, 0)
            bl = (nch - 1) % 2
            pltpu.async_copy(rows_v.at[bl], out_hbm.at[kch_v.at[bl]],
                             ssem).wait()

    return body(indices, newcat)


def kernel(positions, velocities, features, indices,
           new_positions, new_velocities, new_features):
    del positions, velocities, features
    newcat = jnp.concatenate(
        [new_positions, new_velocities, new_features,
         jnp.zeros((_K, _DP - _D), jnp.float32)], axis=1)
    out = _sc_update_gather(indices.astype(jnp.int32), newcat)
    return out[:, :_D]
